# baseline (speedup 1.0000x reference)
import functools

import jax
import jax.numpy as jnp
from jax import lax
from jax.experimental import pallas as pl
from jax.experimental.pallas import tpu as pltpu
from jax.experimental.pallas import tpu_sc as plsc

NUSER = 5000
NITEM = 5000
N = NUSER + NITEM
D = 128
E = 320000
NLAYERS = 3
NC, NS = 2, 16
B = 80
CPT = D // NS
CH = 2560

_mesh = plsc.VectorSubcoreMesh(
    core_axis_name="c", subcore_axis_name="s", num_cores=NC, num_subcores=NS)

_Z16 = functools.partial(jnp.zeros, (16,), jnp.float32)


def _deg_body(src_hbm, dst_hbm, degs_hbm, degd_hbm, idx2, ones_v, zb_v,
              deg_sp):
    c = lax.axis_index("c")
    s = lax.axis_index("s")
    ones16 = jnp.ones((16,), jnp.float32)
    for k in range(B // 16):
        ones_v[pl.ds(16 * k, 16)] = ones16
    for k in range(40):
        zb_v[pl.ds(16 * k, 16)] = _Z16()

    @pl.when(s < 15)
    def _():
        pltpu.sync_copy(zb_v, deg_sp.at[pl.ds(s * 640, 640)])

    @pl.when(s == 15)
    def _():
        pltpu.sync_copy(zb_v.at[pl.ds(0, 400)], deg_sp.at[pl.ds(9600, 400)])

    plsc.subcore_barrier()

    per_tile = E // NS

    def _count(e_hbm):
        pltpu.sync_copy(e_hbm.at[pl.ds(s * per_tile, per_tile)], idx2)

        def body(b, carry):
            pltpu.sync_copy(ones_v, deg_sp.at[idx2.at[pl.ds(b * B, B)]],
                            add=True)
            return carry
        lax.fori_loop(0, per_tile // B, body, 0)

    @pl.when(c == 0)
    def _():
        _count(src_hbm)

    @pl.when(c == 1)
    def _():
        _count(dst_hbm)

    plsc.subcore_barrier()

    def _writeout(out_hbm):
        @pl.when(s < 15)
        def _():
            pltpu.sync_copy(deg_sp.at[pl.ds(s * 640, 640)], zb_v)
            pltpu.sync_copy(zb_v, out_hbm.at[pl.ds(s * 640, 640)])

        @pl.when(s == 15)
        def _():
            pltpu.sync_copy(deg_sp.at[pl.ds(9600, 400)],
                            zb_v.at[pl.ds(0, 400)])
            pltpu.sync_copy(zb_v.at[pl.ds(0, 400)],
                            out_hbm.at[pl.ds(9600, 400)])

    @pl.when(c == 0)
    def _():
        _writeout(degs_hbm)

    @pl.when(c == 1)
    def _():
        _writeout(degd_hbm)


_deg_kernel = pl.kernel(
    _deg_body,
    out_type=(jax.ShapeDtypeStruct((N,), jnp.float32),
              jax.ShapeDtypeStruct((N,), jnp.float32)),
    mesh=_mesh,
    scratch_types=[
        pltpu.VMEM((E // NS,), jnp.int32),
        pltpu.VMEM((B,), jnp.float32),
        pltpu.VMEM((640,), jnp.float32),
        pltpu.VMEM_SHARED((N,), jnp.float32),
    ],
)


def _spmm_body(x_hbm, src_hbm, dst_hbm, out0_hbm, out1_hbm, sidx2, didx2,
               rows3, zb, acc_sp, semg):
    c = lax.axis_index("c")
    s = lax.axis_index("s")

    def zfill(i, carry):
        zb[i // 8, pl.ds((i % 8) * 16, 16)] = _Z16()
        return carry
    lax.fori_loop(0, B * D // 16, zfill, 0)

    def zcopy(k, carry):
        pltpu.sync_copy(zb, acc_sp.at[pl.ds(s * 640 + k * B, B)])
        return carry

    @pl.when(s < 15)
    def _():
        lax.fori_loop(0, 8, zcopy, 0)

    @pl.when(s == 15)
    def _():
        lax.fori_loop(0, 5, zcopy, 0)

    per_tile = E // NC // NS
    nb = per_tile // B
    base = c * (E // NC) + s * per_tile
    pltpu.sync_copy(src_hbm.at[pl.ds(base, per_tile)], sidx2)
    pltpu.sync_copy(dst_hbm.at[pl.ds(base, per_tile)], didx2)

    plsc.subcore_barrier()

    def _gather(b, slot):
        return pltpu.async_copy(x_hbm.at[sidx2.at[pl.ds(b * B, B)]],
                                rows3.at[slot], semg)

    _gather(0, 0)

    def body(b, carry):
        slot = lax.rem(b, 2)
        pltpu.make_async_copy(x_hbm.at[sidx2.at[pl.ds(b * B, B)]],
                              rows3.at[slot], semg).wait()

        @pl.when(b + 1 < nb)
        def _():
            _gather(b + 1, 1 - slot)

        pltpu.sync_copy(rows3.at[slot], acc_sp.at[didx2.at[pl.ds(b * B, B)]],
                        add=True)
        return carry
    lax.fori_loop(0, nb, body, 0)

    plsc.subcore_barrier()

    def _writeout(out_hbm):
        def wchunk(k, carry):
            r0 = s * 640 + k * B
            pltpu.sync_copy(acc_sp.at[pl.ds(r0, B)], rows3.at[0])
            pltpu.sync_copy(rows3.at[0], out_hbm.at[pl.ds(r0, B)])
            return carry

        @pl.when(s < 15)
        def _():
            lax.fori_loop(0, 8, wchunk, 0)

        @pl.when(s == 15)
        def _():
            lax.fori_loop(0, 5, wchunk, 0)

    @pl.when(c == 0)
    def _():
        _writeout(out0_hbm)

    @pl.when(c == 1)
    def _():
        _writeout(out1_hbm)


_spmm_kernel = pl.kernel(
    _spmm_body,
    out_type=(jax.ShapeDtypeStruct((N, D), jnp.float32),
              jax.ShapeDtypeStruct((N, D), jnp.float32)),
    mesh=_mesh,
    scratch_types=[
        pltpu.VMEM((E // NC // NS,), jnp.int32),
        pltpu.VMEM((E // NC // NS,), jnp.int32),
        pltpu.VMEM((2, B, D), jnp.float32),
        pltpu.VMEM((B, D), jnp.float32),
        pltpu.VMEM_SHARED((N, D), jnp.float32),
        pltpu.SemaphoreType.DMA,
    ],
)


def _score_body(ht_hbm, se_hbm, de_hbm, out_hbm, ht_v, sidx, didx, sbuf,
                rowidx, zb, score_sp):
    c = lax.axis_index("c")
    s = lax.axis_index("s")
    iota16 = lax.iota(jnp.int32, 16)

    def zfill(i, carry):
        zb[i // 8, pl.ds((i % 8) * 16, 16)] = _Z16()
        return carry
    lax.fori_loop(0, 160 * D // 16, zfill, 0)

    @pl.when(s < 15)
    def _():
        pltpu.sync_copy(zb, score_sp.at[pl.ds(s * 160, 160)])

    @pl.when(s == 15)
    def _():
        pltpu.sync_copy(zb.at[pl.ds(0, 100)], score_sp.at[pl.ds(2400, 100)])

    pltpu.sync_copy(ht_hbm.at[pl.ds(s * (CPT * N), CPT * N)], ht_v)

    plsc.subcore_barrier()

    nch = E // CH
    ebase = c * E

    def chunk(ch, carry):
        pltpu.sync_copy(se_hbm.at[pl.ds(ebase + ch * CH, CH)], sidx)
        pltpu.sync_copy(de_hbm.at[pl.ds(ebase + ch * CH, CH)], didx)

        def group(g, carry2):
            si = sidx[pl.ds(g * 16, 16)] * CPT
            di = didx[pl.ds(g * 16, 16)] * CPT
            v = None
            for col in range(CPT):
                sv = plsc.load_gather(ht_v, [si + col])
                dv = plsc.load_gather(ht_v, [di + col])
                t = sv * dv
                v = t if v is None else v + t
            sbuf[g // 8, pl.ds((g % 8) * 16, 16)] = v
            return carry2
        lax.fori_loop(0, CH // 16, group, 0, unroll=4)

        rowidx[pl.ds(0, 16)] = iota16 + ch * (CH // D)
        rowidx[pl.ds(4, 16)] = iota16 + (ch * (CH // D) + 4)
        pltpu.sync_copy(sbuf, score_sp.at[rowidx], add=True)
        return carry
    lax.fori_loop(0, nch, chunk, 0)

    plsc.subcore_barrier()

    def wchunk(k, carry):
        r0 = s * 160 + k * B
        pltpu.sync_copy(score_sp.at[pl.ds(r0, B)], zb.at[pl.ds(0, B)])
        pltpu.sync_copy(zb.at[pl.ds(0, B)], out_hbm.at[c, pl.ds(r0, B)])
        return carry

    @pl.when(s < 15)
    def _():
        lax.fori_loop(0, 2, wchunk, 0)

    @pl.when(s == 15)
    def _():
        lax.fori_loop(0, 1, wchunk, 0)

    @pl.when(s == 15)
    def _():
        pltpu.sync_copy(score_sp.at[pl.ds(2480, 20)], zb.at[pl.ds(0, 20)])
        pltpu.sync_copy(zb.at[pl.ds(0, 20)],
                        out_hbm.at[c, pl.ds(2480, 20)])


_score_kernel = pl.kernel(
    _score_body,
    out_type=jax.ShapeDtypeStruct((NC, E // D, D), jnp.float32),
    mesh=_mesh,
    scratch_types=[
        pltpu.VMEM((CPT * N,), jnp.float32),
        pltpu.VMEM((CH,), jnp.int32),
        pltpu.VMEM((CH,), jnp.int32),
        pltpu.VMEM((CH // D, D), jnp.float32),
        pltpu.VMEM((20,), jnp.int32),
        pltpu.VMEM((160, D), jnp.float32),
        pltpu.VMEM_SHARED((E // D, D), jnp.float32),
    ],
    compiler_params=pltpu.CompilerParams(needs_layout_passes=False),
)


def _prep_body(degs_ref, degd_ref, x0_ref, no_ref, ni_ref, xs_ref):
    no = lax.rsqrt(jnp.maximum(degs_ref[...], 1.0))
    ni = lax.rsqrt(jnp.maximum(degd_ref[...], 1.0))
    no_ref[...] = no
    ni_ref[...] = ni
    xs_ref[...] = x0_ref[...] * no


_prep_kernel = pl.pallas_call(
    _prep_body,
    out_shape=(jax.ShapeDtypeStruct((N, 1), jnp.float32),
               jax.ShapeDtypeStruct((N, 1), jnp.float32),
               jax.ShapeDtypeStruct((N, D), jnp.float32)),
)


def _combine_body(coef, p0_ref, p1_ref, ni_ref, no_ref, res_ref, res_out,
                  xn_out):
    emb = (p0_ref[...] + p1_ref[...]) * ni_ref[...]
    res_out[...] = res_ref[...] + emb * coef
    xn_out[...] = emb * no_ref[...]


def _make_combine(coef):
    return pl.pallas_call(
        functools.partial(_combine_body, coef),
        out_shape=(jax.ShapeDtypeStruct((N, D), jnp.float32),
                   jax.ShapeDtypeStruct((N, D), jnp.float32)),
    )


_combine_kernels = [_make_combine(1.0 / (i + 2)) for i in range(NLAYERS - 1)]


def _final_body(p0_ref, p1_ref, ni_ref, res_ref, h_out):
    emb = (p0_ref[...] + p1_ref[...]) * ni_ref[...]
    h_out[...] = res_ref[...] + emb * (1.0 / (NLAYERS + 1))


_final_kernel = pl.pallas_call(
    _final_body,
    out_shape=jax.ShapeDtypeStruct((N, D), jnp.float32),
)


def kernel(U, I, pos_edge_index, neg_edge_index):
    src = pos_edge_index[0]
    dst = pos_edge_index[1]
    degs, degd = _deg_kernel(src, dst)
    x0 = jnp.concatenate([U, I], axis=0)
    no, ni, xs = _prep_kernel(degs.reshape(N, 1), degd.reshape(N, 1), x0)
    res = x0
    for i in range(NLAYERS - 1):
        p0, p1 = _spmm_kernel(xs, src, dst)
        res, xs = _combine_kernels[i](p0, p1, ni, no, res)
    p0, p1 = _spmm_kernel(xs, src, dst)
    h = _final_kernel(p0, p1, ni, res)
    h_t1 = jnp.transpose(h.reshape(N, NS, CPT), (1, 0, 2)).reshape(-1)
    se = jnp.concatenate([src, neg_edge_index[0]])
    de = jnp.concatenate([dst, neg_edge_index[1]])
    scores = _score_kernel(h_t1, se, de).reshape(-1)
    return scores[:E], scores[E:]


# speedup vs baseline: 1.4237x; 1.4237x over previous
import functools

import jax
import jax.numpy as jnp
from jax import lax
from jax.experimental import pallas as pl
from jax.experimental.pallas import tpu as pltpu
from jax.experimental.pallas import tpu_sc as plsc

NUSER = 5000
NITEM = 5000
N = NUSER + NITEM
D = 128
E = 320000
NLAYERS = 3
NC, NS = 2, 16
B = 80

_mesh = plsc.VectorSubcoreMesh(
    core_axis_name="c", subcore_axis_name="s", num_cores=NC, num_subcores=NS)

_Z16 = functools.partial(jnp.zeros, (16,), jnp.float32)


def _lane_shuffle(v, idx):
    return lax.gather(
        v, idx[:, None],
        lax.GatherDimensionNumbers(offset_dims=(), collapsed_slice_dims=(0,),
                                   start_index_map=(0,)),
        slice_sizes=(1,),
        mode=lax.GatherScatterMode.PROMISE_IN_BOUNDS)


def _deg_body(src_hbm, dst_hbm, degs_hbm, degd_hbm, idx2, ones_v, zb_v,
              deg_sp):
    c = lax.axis_index("c")
    s = lax.axis_index("s")
    ones16 = jnp.ones((16,), jnp.float32)
    for k in range(B // 16):
        ones_v[pl.ds(16 * k, 16)] = ones16
    for k in range(40):
        zb_v[pl.ds(16 * k, 16)] = _Z16()

    @pl.when(s < 15)
    def _():
        pltpu.sync_copy(zb_v, deg_sp.at[pl.ds(s * 640, 640)])

    @pl.when(s == 15)
    def _():
        pltpu.sync_copy(zb_v.at[pl.ds(0, 400)], deg_sp.at[pl.ds(9600, 400)])

    plsc.subcore_barrier()

    per_tile = E // NS

    def _count(e_hbm):
        pltpu.sync_copy(e_hbm.at[pl.ds(s * per_tile, per_tile)], idx2)

        def body(b, carry):
            pltpu.sync_copy(ones_v, deg_sp.at[idx2.at[pl.ds(b * B, B)]],
                            add=True)
            return carry
        lax.fori_loop(0, per_tile // B, body, 0)

    @pl.when(c == 0)
    def _():
        _count(src_hbm)

    @pl.when(c == 1)
    def _():
        _count(dst_hbm)

    plsc.subcore_barrier()

    def _writeout(out_hbm):
        @pl.when(s < 15)
        def _():
            pltpu.sync_copy(deg_sp.at[pl.ds(s * 640, 640)], zb_v)
            pltpu.sync_copy(zb_v, out_hbm.at[pl.ds(s * 640, 640)])

        @pl.when(s == 15)
        def _():
            pltpu.sync_copy(deg_sp.at[pl.ds(9600, 400)],
                            zb_v.at[pl.ds(0, 400)])
            pltpu.sync_copy(zb_v.at[pl.ds(0, 400)],
                            out_hbm.at[pl.ds(9600, 400)])

    @pl.when(c == 0)
    def _():
        _writeout(degs_hbm)

    @pl.when(c == 1)
    def _():
        _writeout(degd_hbm)


_deg_kernel = pl.kernel(
    _deg_body,
    out_type=(jax.ShapeDtypeStruct((N,), jnp.float32),
              jax.ShapeDtypeStruct((N,), jnp.float32)),
    mesh=_mesh,
    scratch_types=[
        pltpu.VMEM((E // NS,), jnp.int32),
        pltpu.VMEM((B,), jnp.float32),
        pltpu.VMEM((640,), jnp.float32),
        pltpu.VMEM_SHARED((N,), jnp.float32),
    ],
)


def _spmm_body(x_hbm, src_hbm, dst_hbm, out0_hbm, out1_hbm, sidx2, didx2,
               rows3, zb, acc_sp, semg):
    c = lax.axis_index("c")
    s = lax.axis_index("s")

    def zfill(i, carry):
        zb[i // 8, pl.ds((i % 8) * 16, 16)] = _Z16()
        return carry
    lax.fori_loop(0, B * D // 16, zfill, 0)

    def zcopy(k, carry):
        pltpu.sync_copy(zb, acc_sp.at[pl.ds(s * 640 + k * B, B)])
        return carry

    @pl.when(s < 15)
    def _():
        lax.fori_loop(0, 8, zcopy, 0)

    @pl.when(s == 15)
    def _():
        lax.fori_loop(0, 5, zcopy, 0)

    per_tile = E // NC // NS
    nb = per_tile // B
    base = c * (E // NC) + s * per_tile
    pltpu.sync_copy(src_hbm.at[pl.ds(base, per_tile)], sidx2)
    pltpu.sync_copy(dst_hbm.at[pl.ds(base, per_tile)], didx2)

    plsc.subcore_barrier()

    def _gather(b, slot):
        return pltpu.async_copy(x_hbm.at[sidx2.at[pl.ds(b * B, B)]],
                                rows3.at[slot], semg)

    _gather(0, 0)

    def body(b, carry):
        slot = lax.rem(b, 2)
        pltpu.make_async_copy(x_hbm.at[sidx2.at[pl.ds(b * B, B)]],
                              rows3.at[slot], semg).wait()

        @pl.when(b + 1 < nb)
        def _():
            _gather(b + 1, 1 - slot)

        pltpu.sync_copy(rows3.at[slot], acc_sp.at[didx2.at[pl.ds(b * B, B)]],
                        add=True)
        return carry
    lax.fori_loop(0, nb, body, 0)

    plsc.subcore_barrier()

    def _writeout(out_hbm):
        def wchunk(k, carry):
            r0 = s * 640 + k * B
            pltpu.sync_copy(acc_sp.at[pl.ds(r0, B)], rows3.at[0])
            pltpu.sync_copy(rows3.at[0], out_hbm.at[pl.ds(r0, B)])
            return carry

        @pl.when(s < 15)
        def _():
            lax.fori_loop(0, 8, wchunk, 0)

        @pl.when(s == 15)
        def _():
            lax.fori_loop(0, 5, wchunk, 0)

    @pl.when(c == 0)
    def _():
        _writeout(out0_hbm)

    @pl.when(c == 1)
    def _():
        _writeout(out1_hbm)


_spmm_kernel = pl.kernel(
    _spmm_body,
    out_type=(jax.ShapeDtypeStruct((N, D), jnp.float32),
              jax.ShapeDtypeStruct((N, D), jnp.float32)),
    mesh=_mesh,
    scratch_types=[
        pltpu.VMEM((E // NC // NS,), jnp.int32),
        pltpu.VMEM((E // NC // NS,), jnp.int32),
        pltpu.VMEM((2, B, D), jnp.float32),
        pltpu.VMEM((B, D), jnp.float32),
        pltpu.VMEM_SHARED((N, D), jnp.float32),
        pltpu.SemaphoreType.DMA,
    ],
)


def _score_body(h_hbm, se_hbm, de_hbm, out_hbm, sidx2, didx2, hs3, hd3, ob,
                semg):
    c = lax.axis_index("c")
    s = lax.axis_index("s")
    w = s * NC + c
    iota16 = lax.iota(jnp.int32, 16)
    per_w = 2 * E // (NC * NS)
    nb = per_w // B

    base = w * per_w
    pltpu.sync_copy(se_hbm.at[pl.ds(base, per_w)], sidx2)
    pltpu.sync_copy(de_hbm.at[pl.ds(base, per_w)], didx2)

    def _gather(b, slot):
        return (pltpu.async_copy(h_hbm.at[sidx2.at[pl.ds(b * B, B)]],
                                 hs3.at[slot], semg),
                pltpu.async_copy(h_hbm.at[didx2.at[pl.ds(b * B, B)]],
                                 hd3.at[slot], semg))

    _gather(0, 0)

    def body(b, carry):
        slot = lax.rem(b, 2)
        pltpu.make_async_copy(h_hbm.at[sidx2.at[pl.ds(b * B, B)]],
                              hs3.at[slot], semg).wait()
        pltpu.make_async_copy(h_hbm.at[didx2.at[pl.ds(b * B, B)]],
                              hd3.at[slot], semg).wait()

        @pl.when(b + 1 < nb)
        def _():
            _gather(b + 1, 1 - slot)

        for g in range(B // 16):
            vs = []
            for e in range(16):
                row = 16 * g + e
                v = (hs3[slot, row, pl.ds(0, 16)] *
                     hd3[slot, row, pl.ds(0, 16)])
                for k in range(1, D // 16):
                    v = v + (hs3[slot, row, pl.ds(16 * k, 16)] *
                             hd3[slot, row, pl.ds(16 * k, 16)])
                vs.append(v)
            for sh in (1, 2, 4, 8):
                nxt = []
                for i in range(len(vs) // 2):
                    a, b2 = vs[2 * i], vs[2 * i + 1]
                    nxt.append(jnp.where(
                        (iota16 & sh) == 0,
                        a + _lane_shuffle(a, iota16 ^ sh),
                        b2 + _lane_shuffle(b2, iota16 ^ sh)))
                vs = nxt
            ob[pl.ds(16 * g, 16)] = vs[0]
        pltpu.sync_copy(ob, out_hbm.at[pl.ds(base + b * B, B)])
        return carry
    lax.fori_loop(0, nb, body, 0)


_score_kernel = pl.kernel(
    _score_body,
    out_type=jax.ShapeDtypeStruct((2 * E,), jnp.float32),
    mesh=_mesh,
    scratch_types=[
        pltpu.VMEM((2 * E // (NC * NS),), jnp.int32),
        pltpu.VMEM((2 * E // (NC * NS),), jnp.int32),
        pltpu.VMEM((2, B, D), jnp.float32),
        pltpu.VMEM((2, B, D), jnp.float32),
        pltpu.VMEM((B,), jnp.float32),
        pltpu.SemaphoreType.DMA,
    ],
)


def _prep_body(degs_ref, degd_ref, x0_ref, no_ref, ni_ref, xs_ref):
    no = lax.rsqrt(jnp.maximum(degs_ref[...], 1.0))
    ni = lax.rsqrt(jnp.maximum(degd_ref[...], 1.0))
    no_ref[...] = no
    ni_ref[...] = ni
    xs_ref[...] = x0_ref[...] * no


_prep_kernel = pl.pallas_call(
    _prep_body,
    out_shape=(jax.ShapeDtypeStruct((N, 1), jnp.float32),
               jax.ShapeDtypeStruct((N, 1), jnp.float32),
               jax.ShapeDtypeStruct((N, D), jnp.float32)),
)


def _combine_body(coef, p0_ref, p1_ref, ni_ref, no_ref, res_ref, res_out,
                  xn_out):
    emb = (p0_ref[...] + p1_ref[...]) * ni_ref[...]
    res_out[...] = res_ref[...] + emb * coef
    xn_out[...] = emb * no_ref[...]


def _make_combine(coef):
    return pl.pallas_call(
        functools.partial(_combine_body, coef),
        out_shape=(jax.ShapeDtypeStruct((N, D), jnp.float32),
                   jax.ShapeDtypeStruct((N, D), jnp.float32)),
    )


_combine_kernels = [_make_combine(1.0 / (i + 2)) for i in range(NLAYERS)]


def kernel(U, I, pos_edge_index, neg_edge_index):
    src = pos_edge_index[0]
    dst = pos_edge_index[1]
    degs, degd = _deg_kernel(src, dst)
    x0 = jnp.concatenate([U, I], axis=0)
    no, ni, xs = _prep_kernel(degs.reshape(N, 1), degd.reshape(N, 1), x0)
    res = x0
    for i in range(NLAYERS):
        p0, p1 = _spmm_kernel(xs, src, dst)
        res, xs = _combine_kernels[i](p0, p1, ni, no, res)
    se = jnp.concatenate([src, neg_edge_index[0]])
    de = jnp.concatenate([dst, neg_edge_index[1]])
    scores = _score_kernel(res, se, de)
    return scores[:E], scores[E:]
